# baseline (speedup 1.0000x reference)
import jax
import jax.numpy as jnp
from jax import lax
from jax.experimental import pallas as pl
from jax.experimental.pallas import tpu as pltpu, tpu_sc as plsc

DIM = 128
CTX = 200
WPC = 5
ROWS = CTX // WPC
NLANES = 16
NCH = DIM // NLANES


def _sc_body(nb_hbm, pt_hbm, ed_hbm, nd_hbm, ent_hbm, rel_hbm,
             score_hbm, loss_hbm,
             idx_v, rows_v, nidx_v, si_v, svec_v, sall_v, out_v, shared, sem):
    sid = lax.axis_index("s")

    def do_partial(ctx_idx_hbm, table_hbm, base_sid):
        j = sid - base_sid
        pltpu.sync_copy(ctx_idx_hbm.at[pl.ds(j * ROWS, ROWS)], idx_v)
        pltpu.async_copy(table_hbm.at[idx_v], rows_v, sem).wait()
        def rbody(i, acc):
            return tuple(acc[k] + rows_v[i, pl.ds(NLANES * k, NLANES)]
                         for k in range(NCH))
        acc = lax.fori_loop(
            0, ROWS, rbody,
            tuple(jnp.zeros((NLANES,), jnp.float32) for _ in range(NCH)))
        for k in range(NCH):
            svec_v[pl.ds(NLANES * k, NLANES)] = acc[k]
        pltpu.sync_copy(svec_v, shared.at[sid])

    @pl.when(sid < WPC)
    def _():
        do_partial(nb_hbm, ent_hbm, 0)

    @pl.when((sid >= WPC) & (sid < 2 * WPC))
    def _():
        do_partial(pt_hbm, ent_hbm, WPC)

    @pl.when((sid >= 2 * WPC) & (sid < 3 * WPC))
    def _():
        do_partial(ed_hbm, rel_hbm, 2 * WPC)

    @pl.when(sid == 15)
    def _():
        pltpu.sync_copy(nd_hbm, nidx_v)
        pltpu.async_copy(ent_hbm.at[nidx_v], si_v, sem).wait()

    plsc.subcore_barrier()

    @pl.when(sid == 15)
    def _():
        pltpu.sync_copy(shared, sall_v)
        def cbody(k, carry):
            nums, dens = carry
            sik = si_v[0, pl.ds(NLANES * k, NLANES)]
            new_nums, new_dens = [], []
            for c in range(3):
                def jbody(j, s):
                    return s + sall_v[c * WPC + j, pl.ds(NLANES * k, NLANES)]
                sck = lax.fori_loop(0, WPC, jbody,
                                    jnp.zeros((NLANES,), jnp.float32))
                new_nums.append(nums[c] + sik * sck)
                new_dens.append(dens[c] + sck * sck)
            return tuple(new_nums), tuple(new_dens)
        zeros3 = tuple(jnp.zeros((NLANES,), jnp.float32) for _ in range(3))
        nums, dens = lax.fori_loop(0, NCH, cbody, (zeros3, zeros3))
        ps = []
        for c in range(3):
            num_v = jnp.full((NLANES,), jnp.sum(nums[c]), jnp.float32)
            den_v = jnp.full((NLANES,), jnp.sum(dens[c]), jnp.float32)
            ps.append(num_v / den_v)
        score_v = ps[0] + jnp.float32(0.1) * ps[1] + jnp.float32(0.1) * ps[2]
        loss_v = jnp.float32(1.2) - score_v
        out_v[0, :] = score_v
        out_v[1, :] = loss_v
        pltpu.sync_copy(out_v.at[0, pl.ds(0, 1)], score_hbm.at[0])
        pltpu.sync_copy(out_v.at[1, pl.ds(0, 1)], loss_hbm.at[0])


@jax.jit
def _gake_sc(nb_idx, pt_idx, ed_idx, nd_idx, ent_table, rel_table):
    mesh = plsc.VectorSubcoreMesh(core_axis_name="c", subcore_axis_name="s",
                                  num_cores=1)
    return pl.kernel(
        _sc_body,
        out_type=(jax.ShapeDtypeStruct((1, 1), jnp.float32),
                  jax.ShapeDtypeStruct((1, 1), jnp.float32)),
        mesh=mesh,
        compiler_params=pltpu.CompilerParams(needs_layout_passes=False, skip_device_barrier=True),
        scratch_types=[
            pltpu.VMEM((ROWS,), jnp.int32),
            pltpu.VMEM((ROWS, DIM), jnp.float32),
            pltpu.VMEM((1,), jnp.int32),
            pltpu.VMEM((1, DIM), jnp.float32),
            pltpu.VMEM((DIM,), jnp.float32),
            pltpu.VMEM((NLANES, DIM), jnp.float32),
            pltpu.VMEM((2, NLANES), jnp.float32),
            pltpu.VMEM_SHARED((NLANES, DIM), jnp.float32),
            pltpu.SemaphoreType.DMA,
        ],
        name="gake_sc",
    )(nb_idx, pt_idx, ed_idx, nd_idx, ent_table, rel_table)


@jax.jit
def kernel(node_id, neighbor_ids, path_ids, edge_ids, ent_table, rel_table):
    nb = neighbor_ids.astype(jnp.int32)
    pt = path_ids.astype(jnp.int32)
    ed = edge_ids.astype(jnp.int32)
    nd = node_id.astype(jnp.int32)
    score, loss = _gake_sc(nb, pt, ed, nd, ent_table, rel_table)
    return (score, loss)


# speedup vs baseline: 1.0038x; 1.0038x over previous
import jax
import jax.numpy as jnp
from jax import lax
from jax.experimental import pallas as pl
from jax.experimental.pallas import tpu as pltpu, tpu_sc as plsc

DIM = 128
CTX = 200
WPC = 5
ROWS = CTX // WPC
NLANES = 16
NCH = DIM // NLANES


def _sc_body(nb_hbm, pt_hbm, ed_hbm, nd_hbm, ent_hbm, rel_hbm,
             score_hbm, loss_hbm,
             idx_v, rows_v, nidx_v, si_v, svec_v, sall_v, out_v, shared, sem):
    sid = lax.axis_index("s")

    def do_partial(ctx_idx_hbm, table_hbm, base_sid):
        j = sid - base_sid
        pltpu.sync_copy(ctx_idx_hbm.at[pl.ds(j * ROWS, ROWS)], idx_v)
        pltpu.async_copy(table_hbm.at[idx_v], rows_v, sem).wait()
        def rbody(i, acc):
            return tuple(acc[k] + rows_v[i, pl.ds(NLANES * k, NLANES)]
                         for k in range(NCH))
        acc = lax.fori_loop(
            0, ROWS, rbody,
            tuple(jnp.zeros((NLANES,), jnp.float32) for _ in range(NCH)))
        for k in range(NCH):
            svec_v[pl.ds(NLANES * k, NLANES)] = acc[k]
        pltpu.sync_copy(svec_v, shared.at[sid])

    @pl.when(sid < WPC)
    def _():
        do_partial(nb_hbm, ent_hbm, 0)

    @pl.when((sid >= WPC) & (sid < 2 * WPC))
    def _():
        do_partial(pt_hbm, ent_hbm, WPC)

    @pl.when((sid >= 2 * WPC) & (sid < 3 * WPC))
    def _():
        do_partial(ed_hbm, rel_hbm, 2 * WPC)

    @pl.when(sid == 15)
    def _():
        pltpu.sync_copy(nd_hbm, nidx_v)
        pltpu.async_copy(ent_hbm.at[nidx_v], si_v, sem).wait()

    plsc.subcore_barrier()

    @pl.when(sid == 15)
    def _():
        pltpu.sync_copy(shared, sall_v)
        def cbody(k, carry):
            nums, dens = carry
            sik = si_v[0, pl.ds(NLANES * k, NLANES)]
            new_nums, new_dens = [], []
            for c in range(3):
                def jbody(j, s):
                    return s + sall_v[c * WPC + j, pl.ds(NLANES * k, NLANES)]
                sck = lax.fori_loop(0, WPC, jbody,
                                    jnp.zeros((NLANES,), jnp.float32))
                new_nums.append(nums[c] + sik * sck)
                new_dens.append(dens[c] + sck * sck)
            return tuple(new_nums), tuple(new_dens)
        zeros3 = tuple(jnp.zeros((NLANES,), jnp.float32) for _ in range(3))
        nums, dens = lax.fori_loop(0, NCH, cbody, (zeros3, zeros3))
        ps = []
        for c in range(3):
            num_v = jnp.full((NLANES,), jnp.sum(nums[c]), jnp.float32)
            den_v = jnp.full((NLANES,), jnp.sum(dens[c]), jnp.float32)
            ps.append(num_v / den_v)
        score_v = ps[0] + jnp.float32(0.1) * ps[1] + jnp.float32(0.1) * ps[2]
        loss_v = jnp.float32(1.2) - score_v
        out_v[0, :] = score_v
        out_v[1, :] = loss_v
        pltpu.sync_copy(out_v.at[0, pl.ds(0, 1)], score_hbm.at[0])
        pltpu.sync_copy(out_v.at[1, pl.ds(0, 1)], loss_hbm.at[0])


@jax.jit
def _gake_sc(nb_idx, pt_idx, ed_idx, nd_idx, ent_table, rel_table):
    mesh = plsc.VectorSubcoreMesh(core_axis_name="c", subcore_axis_name="s",
                                  num_cores=1)
    return pl.kernel(
        _sc_body,
        out_type=(jax.ShapeDtypeStruct((1, 1), jnp.float32),
                  jax.ShapeDtypeStruct((1, 1), jnp.float32)),
        mesh=mesh,
        compiler_params=pltpu.CompilerParams(needs_layout_passes=False),
        scratch_types=[
            pltpu.VMEM((ROWS,), jnp.int32),
            pltpu.VMEM((ROWS, DIM), jnp.float32),
            pltpu.VMEM((1,), jnp.int32),
            pltpu.VMEM((1, DIM), jnp.float32),
            pltpu.VMEM((DIM,), jnp.float32),
            pltpu.VMEM((NLANES, DIM), jnp.float32),
            pltpu.VMEM((2, NLANES), jnp.float32),
            pltpu.VMEM_SHARED((NLANES, DIM), jnp.float32),
            pltpu.SemaphoreType.DMA,
        ],
        name="gake_sc",
    )(nb_idx, pt_idx, ed_idx, nd_idx, ent_table, rel_table)


@jax.jit
def kernel(node_id, neighbor_ids, path_ids, edge_ids, ent_table, rel_table):
    nb = neighbor_ids.astype(jnp.int32)
    pt = path_ids.astype(jnp.int32)
    ed = edge_ids.astype(jnp.int32)
    nd = node_id.astype(jnp.int32)
    score, loss = _gake_sc(nb, pt, ed, nd, ent_table, rel_table)
    return (score, loss)


# speedup vs baseline: 1.0059x; 1.0020x over previous
import jax
import jax.numpy as jnp
from jax import lax
from jax.experimental import pallas as pl
from jax.experimental.pallas import tpu as pltpu, tpu_sc as plsc

DIM = 128
CTX = 200
WPC = 5
ROWS = CTX // WPC
NLANES = 16
NCH = DIM // NLANES


def _sc_body(nb_hbm, pt_hbm, ed_hbm, nd_hbm, ent_hbm, rel_hbm,
             score_hbm, loss_hbm,
             idx_v, rows_v, nidx_v, si_v, svec_v, sall_v, out_v, shared, sem):
    sid = lax.axis_index("s")

    def do_partial(ctx_idx_hbm, table_hbm, base_sid):
        j = sid - base_sid
        pltpu.sync_copy(ctx_idx_hbm.at[pl.ds(j * ROWS, ROWS)], idx_v)
        pltpu.async_copy(table_hbm.at[idx_v], rows_v, sem).wait()
        def rbody(i, acc):
            return tuple(acc[k] + rows_v[i, pl.ds(NLANES * k, NLANES)]
                         for k in range(NCH))
        acc = lax.fori_loop(
            0, ROWS, rbody,
            tuple(jnp.zeros((NLANES,), jnp.float32) for _ in range(NCH)))
        for k in range(NCH):
            svec_v[pl.ds(NLANES * k, NLANES)] = acc[k]
        pltpu.sync_copy(svec_v, shared.at[sid])

    @pl.when(sid < WPC)
    def _():
        do_partial(nb_hbm, ent_hbm, 0)

    @pl.when((sid >= WPC) & (sid < 2 * WPC))
    def _():
        do_partial(pt_hbm, ent_hbm, WPC)

    @pl.when((sid >= 2 * WPC) & (sid < 3 * WPC))
    def _():
        do_partial(ed_hbm, rel_hbm, 2 * WPC)

    @pl.when(sid == 15)
    def _():
        pltpu.sync_copy(nd_hbm, nidx_v)
        pltpu.async_copy(ent_hbm.at[nidx_v], si_v, sem).wait()

    plsc.subcore_barrier()

    @pl.when(sid == 15)
    def _():
        pltpu.sync_copy(shared, sall_v)
        def cbody(k, carry):
            nums, dens = carry
            sik = si_v[0, pl.ds(NLANES * k, NLANES)]
            new_nums, new_dens = [], []
            for c in range(3):
                def jbody(j, s):
                    return s + sall_v[c * WPC + j, pl.ds(NLANES * k, NLANES)]
                sck = lax.fori_loop(0, WPC, jbody,
                                    jnp.zeros((NLANES,), jnp.float32))
                new_nums.append(nums[c] + sik * sck)
                new_dens.append(dens[c] + sck * sck)
            return tuple(new_nums), tuple(new_dens)
        zeros3 = tuple(jnp.zeros((NLANES,), jnp.float32) for _ in range(3))
        nums, dens = lax.fori_loop(0, NCH, cbody, (zeros3, zeros3))
        ps = []
        for c in range(3):
            num_v = jnp.full((NLANES,), jnp.sum(nums[c]), jnp.float32)
            den_v = jnp.full((NLANES,), jnp.sum(dens[c]), jnp.float32)
            ps.append(num_v / den_v)
        score_v = ps[0] + jnp.float32(0.1) * ps[1] + jnp.float32(0.1) * ps[2]
        loss_v = jnp.float32(1.2) - score_v
        out_v[0, :] = score_v
        out_v[1, :] = loss_v
        c1 = pltpu.async_copy(out_v.at[0, pl.ds(0, 1)], score_hbm.at[0], sem)
        c2 = pltpu.async_copy(out_v.at[1, pl.ds(0, 1)], loss_hbm.at[0], sem)
        c1.wait()
        c2.wait()


@jax.jit
def _gake_sc(nb_idx, pt_idx, ed_idx, nd_idx, ent_table, rel_table):
    mesh = plsc.VectorSubcoreMesh(core_axis_name="c", subcore_axis_name="s",
                                  num_cores=1)
    return pl.kernel(
        _sc_body,
        out_type=(jax.ShapeDtypeStruct((1, 1), jnp.float32),
                  jax.ShapeDtypeStruct((1, 1), jnp.float32)),
        mesh=mesh,
        compiler_params=pltpu.CompilerParams(needs_layout_passes=False),
        scratch_types=[
            pltpu.VMEM((ROWS,), jnp.int32),
            pltpu.VMEM((ROWS, DIM), jnp.float32),
            pltpu.VMEM((1,), jnp.int32),
            pltpu.VMEM((1, DIM), jnp.float32),
            pltpu.VMEM((DIM,), jnp.float32),
            pltpu.VMEM((NLANES, DIM), jnp.float32),
            pltpu.VMEM((2, NLANES), jnp.float32),
            pltpu.VMEM_SHARED((NLANES, DIM), jnp.float32),
            pltpu.SemaphoreType.DMA,
        ],
        name="gake_sc",
    )(nb_idx, pt_idx, ed_idx, nd_idx, ent_table, rel_table)


@jax.jit
def kernel(node_id, neighbor_ids, path_ids, edge_ids, ent_table, rel_table):
    nb = neighbor_ids.astype(jnp.int32)
    pt = path_ids.astype(jnp.int32)
    ed = edge_ids.astype(jnp.int32)
    nd = node_id.astype(jnp.int32)
    score, loss = _gake_sc(nb, pt, ed, nd, ent_table, rel_table)
    return (score, loss)
